# baseline (speedup 1.0000x reference)
import jax
import jax.numpy as jnp
from jax import lax
from jax.experimental import pallas as pl
from jax.experimental.pallas import tpu as pltpu
from jax.experimental.pallas import tpu_sc as plsc

_NUM_USERS = 60000
_NUM_ITEMS = 40000
_N = _NUM_USERS + _NUM_ITEMS
_NPAD = 102400
_D = 64
_DC = 16
_NCHUNK = 4
_NLAYERS = 3

_E = 1_000_000
_B = 128
_NS = 16
_SB = 16
_NSUP = 31
_NBLK = _SB * _NSUP
_EPT = _B * _NBLK
_EPAD = _EPT * _NS

_RPT = _NPAD // _NS
_MSLAB = 100
_SPT = _RPT // _MSLAB

_f32 = jnp.float32


def _body(ueh, ieh, srch, dsth2, wh, zh,
          uoh, ioh, e0, e1, e2, e3,
          acc, srcv, dstv2, wv, rows0, rows1, rows2, rows3,
          prebuf, mo,
          gsem0, gsem1, gsem2, gsem3, ssem0, ssem1, ssem2, ssem3):
    cid = lax.axis_index("c")
    sid = lax.axis_index("s")
    tbase = sid * _EPT
    row0 = sid * _RPT

    tables = [e0, e1, e2, e3]
    rbufs = (rows0, rows1, rows2, rows3)
    gsems = (gsem0, gsem1, gsem2, gsem3)
    ssems = (ssem0, ssem1, ssem2, ssem3)

    ku = jnp.clip(600 - _SPT * sid, 0, _SPT)
    ke = jnp.clip(1000 - _SPT * sid, 0, _SPT)
    ccol = 2 * cid * _DC

    def _pre(k, src_tab, roff):
        r = row0 + k * _MSLAB
        pltpu.sync_copy(
            src_tab.at[pl.ds(r + roff, _MSLAB), pl.ds(ccol, 2 * _DC)],
            prebuf.at[:, pl.ds(0, 2 * _DC)])
        for st in range(2):
            cc = 2 * cid + st
            pltpu.sync_copy(
                prebuf.at[:, pl.ds(st * _DC, _DC)],
                e0.at[pl.ds(cc * _NPAD + r, _MSLAB)])

    def _pre_u(k, c):
        _pre(k, ueh, 0)
        return c

    def _pre_i(k, c):
        _pre(k, ieh, -_NUM_USERS)
        return c

    lax.fori_loop(0, ku, _pre_u, 0)
    lax.fori_loop(ku, ke, _pre_i, 0)
    plsc.subcore_barrier()

    def _edge_pass(src_tab, ccoff):
        tab = src_tab.at[pl.ds(ccoff, _NPAD)]

        def _sup(s, carry):
            off = tbase + s * (_SB * _B)
            blk0 = sid * _NBLK + s * _SB
            pltpu.sync_copy(srch.at[pl.ds(off, _SB * _B)], srcv)
            pltpu.sync_copy(dsth2.at[pl.ds(blk0, _SB)], dstv2)
            pltpu.sync_copy(wh.at[pl.ds(off, _SB * _B)], wv)

            gd = [None] * _SB
            sd = [None] * _SB
            for j in range(3):
                gd[j] = pltpu.async_copy(
                    tab.at[srcv.at[pl.ds(j * _B, _B)]], rbufs[j], gsems[j])
            for j in range(_SB):
                b = j % 4
                buf = rbufs[b]
                gd[j].wait()

                def _mul(i, c2, j=j, buf=buf):
                    widx = jnp.full((_DC,), j * _B + i, jnp.int32)
                    wsplat = plsc.load_gather(wv, [widx])
                    buf[i, :] = buf[i, :] * wsplat
                    return c2

                lax.fori_loop(0, _B, _mul, 0, unroll=8)
                sd[j] = pltpu.async_copy(
                    buf, acc.at[dstv2.at[j]], ssems[b], add=True)
                nj = j + 3
                if nj < _SB:
                    if j >= 1:
                        sd[j - 1].wait()
                    gd[nj] = pltpu.async_copy(
                        tab.at[srcv.at[pl.ds(nj * _B, _B)]],
                        rbufs[nj % 4], gsems[nj % 4])
            for j in range(_SB - 3, _SB):
                sd[j].wait()
            return carry

        lax.fori_loop(0, _NSUP, _sup, 0)

    for layer in range(_NLAYERS):
        src_tab = tables[layer]
        dst_tab = tables[layer + 1]

        def _pass(step, carry, src_tab=src_tab, dst_tab=dst_tab):
            ccoff = (2 * cid + step) * _NPAD
            pltpu.sync_copy(zh.at[pl.ds(row0, _RPT)],
                            acc.at[pl.ds(row0, _RPT)])
            plsc.subcore_barrier()
            _edge_pass(src_tab, ccoff)
            plsc.subcore_barrier()
            pltpu.sync_copy(acc.at[pl.ds(row0, _RPT)],
                            dst_tab.at[pl.ds(ccoff + row0, _RPT)])
            plsc.subcore_barrier()
            return carry

        lax.fori_loop(0, 2, _pass, 0)

    quarter = _f32(0.25)

    def _mean(step, carry):
        cc = 2 * cid + step
        ccoff = cc * _NPAD
        col = cc * _DC

        def _mk(k, out, roff):
            r = row0 + k * _MSLAB
            for l in range(4):
                pltpu.sync_copy(tables[l].at[pl.ds(ccoff + r, _MSLAB)],
                                prebuf.at[:, pl.ds(l * _DC, _DC)])

            def _mb(i, c3):
                mo[i, :] = (prebuf[i, pl.ds(0, _DC)]
                            + prebuf[i, pl.ds(_DC, _DC)]
                            + prebuf[i, pl.ds(2 * _DC, _DC)]
                            + prebuf[i, pl.ds(3 * _DC, _DC)]) * quarter
                return c3

            lax.fori_loop(0, _MSLAB, _mb, 0, unroll=5)
            pltpu.sync_copy(
                mo, out.at[pl.ds(r + roff, _MSLAB), pl.ds(col, _DC)])

        def _mk_u(k, c2):
            _mk(k, uoh, 0)
            return c2

        def _mk_i(k, c2):
            _mk(k, ioh, -_NUM_USERS)
            return c2

        lax.fori_loop(0, ku, _mk_u, 0)
        lax.fori_loop(ku, ke, _mk_i, 0)
        return carry

    lax.fori_loop(0, 2, _mean, 0)


_tab_t = jax.ShapeDtypeStruct((_NCHUNK * _NPAD, _DC), _f32)

_gcn = pl.kernel(
    _body,
    out_type=(jax.ShapeDtypeStruct((_NUM_USERS, _D), _f32),
              jax.ShapeDtypeStruct((_NUM_ITEMS, _D), _f32),
              _tab_t, _tab_t, _tab_t, _tab_t),
    mesh=plsc.VectorSubcoreMesh(core_axis_name="c", subcore_axis_name="s"),
    compiler_params=pltpu.CompilerParams(use_tc_tiling_on_sc=False,
                                         needs_layout_passes=False),
    scratch_types=[
        pltpu.VMEM_SHARED((_NPAD, _DC), _f32),
        pltpu.VMEM((_SB * _B,), jnp.int32),
        pltpu.VMEM((_SB, _B), jnp.int32),
        pltpu.VMEM((_SB * _B,), _f32),
        pltpu.VMEM((_B, _DC), _f32),
        pltpu.VMEM((_B, _DC), _f32),
        pltpu.VMEM((_B, _DC), _f32),
        pltpu.VMEM((_B, _DC), _f32),
        pltpu.VMEM((_MSLAB, _D), _f32),
        pltpu.VMEM((_MSLAB, _DC), _f32),
        pltpu.SemaphoreType.DMA,
        pltpu.SemaphoreType.DMA,
        pltpu.SemaphoreType.DMA,
        pltpu.SemaphoreType.DMA,
        pltpu.SemaphoreType.DMA,
        pltpu.SemaphoreType.DMA,
        pltpu.SemaphoreType.DMA,
        pltpu.SemaphoreType.DMA,
    ],
)


@jax.jit
def kernel(user_emb, item_emb, edge_weight, edge_index):
    src = edge_index[0].astype(jnp.int32)
    dst = edge_index[1].astype(jnp.int32)
    w = edge_weight.astype(_f32)
    pad = _EPAD - _E
    src = jnp.concatenate([src, jnp.zeros((pad,), jnp.int32)])
    dst = jnp.concatenate([dst, jnp.zeros((pad,), jnp.int32)])
    dst2 = dst.reshape(_EPAD // _B, _B)
    w = jnp.concatenate([w, jnp.zeros((pad,), _f32)])
    zh = jnp.zeros((_NPAD, _DC), _f32)
    res = _gcn(user_emb, item_emb, src, dst2, w, zh)
    return res[0], res[1]


# speedup vs baseline: 1.0079x; 1.0079x over previous
import jax
import jax.numpy as jnp
from jax import lax
from jax.experimental import pallas as pl
from jax.experimental.pallas import tpu as pltpu
from jax.experimental.pallas import tpu_sc as plsc

_NUM_USERS = 60000
_NUM_ITEMS = 40000
_N = _NUM_USERS + _NUM_ITEMS
_NPAD = 102400
_D = 64
_DC = 16
_NCHUNK = 4
_NLAYERS = 3

_E = 1_000_000
_B = 128
_NS = 16
_SB = 16
_NSUP = 31
_NBLK = _SB * _NSUP
_EPT = _B * _NBLK
_EPAD = _EPT * _NS

_RPT = _NPAD // _NS
_MSLAB = 100
_SPT = _RPT // _MSLAB

_f32 = jnp.float32


def _body(ueh, ieh, srch, dsth2, wh, zh,
          uoh, ioh, e0, e1, e2, e3,
          acc, srcv, dstv2, wv, rows0, rows1, rows2, rows3,
          prebuf, mo,
          gsem0, gsem1, gsem2, gsem3, ssem0, ssem1, ssem2, ssem3):
    cid = lax.axis_index("c")
    sid = lax.axis_index("s")
    tbase = sid * _EPT
    row0 = sid * _RPT

    tables = [e0, e1, e2, e3]
    rbufs = (rows0, rows1, rows2, rows3)
    gsems = (gsem0, gsem1, gsem2, gsem3)
    ssems = (ssem0, ssem1, ssem2, ssem3)

    ku = jnp.clip(600 - _SPT * sid, 0, _SPT)
    ke = jnp.clip(1000 - _SPT * sid, 0, _SPT)
    ccol = 2 * cid * _DC

    def _pre(k, src_tab, roff):
        r = row0 + k * _MSLAB
        pltpu.sync_copy(
            src_tab.at[pl.ds(r + roff, _MSLAB), pl.ds(ccol, 2 * _DC)],
            prebuf.at[:, pl.ds(0, 2 * _DC)])
        for st in range(2):
            cc = 2 * cid + st
            pltpu.sync_copy(
                prebuf.at[:, pl.ds(st * _DC, _DC)],
                e0.at[pl.ds(cc * _NPAD + r, _MSLAB)])

    def _pre_u(k, c):
        _pre(k, ueh, 0)
        return c

    def _pre_i(k, c):
        _pre(k, ieh, -_NUM_USERS)
        return c

    lax.fori_loop(0, ku, _pre_u, 0)
    lax.fori_loop(ku, ke, _pre_i, 0)
    plsc.subcore_barrier()

    def _edge_pass(src_tab, ccoff):
        tab = src_tab.at[pl.ds(ccoff, _NPAD)]

        def _sup(s, carry):
            off = tbase + s * (_SB * _B)
            blk0 = sid * _NBLK + s * _SB
            pltpu.sync_copy(srch.at[pl.ds(off, _SB * _B)], srcv)
            pltpu.sync_copy(dsth2.at[pl.ds(blk0, _SB)], dstv2)
            pltpu.sync_copy(wh.at[pl.ds(off, _SB * _B)], wv)

            gd = [None] * _SB
            sd = [None] * _SB
            for j in range(3):
                gd[j] = pltpu.async_copy(
                    tab.at[srcv.at[pl.ds(j * _B, _B)]], rbufs[j], gsems[j])
            for j in range(_SB):
                b = j % 4
                buf = rbufs[b]
                gd[j].wait()

                def _mul(i, c2, j=j, buf=buf):
                    widx = jnp.full((_DC,), j * _B + i, jnp.int32)
                    wsplat = plsc.load_gather(wv, [widx])
                    buf[i, :] = buf[i, :] * wsplat
                    return c2

                lax.fori_loop(0, _B, _mul, 0, unroll=4)
                sd[j] = pltpu.async_copy(
                    buf, acc.at[dstv2.at[j]], ssems[b], add=True)
                nj = j + 3
                if nj < _SB:
                    if j >= 1:
                        sd[j - 1].wait()
                    gd[nj] = pltpu.async_copy(
                        tab.at[srcv.at[pl.ds(nj * _B, _B)]],
                        rbufs[nj % 4], gsems[nj % 4])
            for j in range(_SB - 3, _SB):
                sd[j].wait()
            return carry

        lax.fori_loop(0, _NSUP, _sup, 0)

    for layer in range(_NLAYERS):
        src_tab = tables[layer]
        dst_tab = tables[layer + 1]

        def _pass(step, carry, src_tab=src_tab, dst_tab=dst_tab):
            ccoff = (2 * cid + step) * _NPAD
            pltpu.sync_copy(zh.at[pl.ds(row0, _RPT)],
                            acc.at[pl.ds(row0, _RPT)])
            plsc.subcore_barrier()
            _edge_pass(src_tab, ccoff)
            plsc.subcore_barrier()
            pltpu.sync_copy(acc.at[pl.ds(row0, _RPT)],
                            dst_tab.at[pl.ds(ccoff + row0, _RPT)])
            plsc.subcore_barrier()
            return carry

        lax.fori_loop(0, 2, _pass, 0)

    quarter = _f32(0.25)

    def _mean(step, carry):
        cc = 2 * cid + step
        ccoff = cc * _NPAD
        col = cc * _DC

        def _mk(k, out, roff):
            r = row0 + k * _MSLAB
            for l in range(4):
                pltpu.sync_copy(tables[l].at[pl.ds(ccoff + r, _MSLAB)],
                                prebuf.at[:, pl.ds(l * _DC, _DC)])

            def _mb(i, c3):
                mo[i, :] = (prebuf[i, pl.ds(0, _DC)]
                            + prebuf[i, pl.ds(_DC, _DC)]
                            + prebuf[i, pl.ds(2 * _DC, _DC)]
                            + prebuf[i, pl.ds(3 * _DC, _DC)]) * quarter
                return c3

            lax.fori_loop(0, _MSLAB, _mb, 0, unroll=4)
            pltpu.sync_copy(
                mo, out.at[pl.ds(r + roff, _MSLAB), pl.ds(col, _DC)])

        def _mk_u(k, c2):
            _mk(k, uoh, 0)
            return c2

        def _mk_i(k, c2):
            _mk(k, ioh, -_NUM_USERS)
            return c2

        lax.fori_loop(0, ku, _mk_u, 0)
        lax.fori_loop(ku, ke, _mk_i, 0)
        return carry

    lax.fori_loop(0, 2, _mean, 0)


_tab_t = jax.ShapeDtypeStruct((_NCHUNK * _NPAD, _DC), _f32)

_gcn = pl.kernel(
    _body,
    out_type=(jax.ShapeDtypeStruct((_NUM_USERS, _D), _f32),
              jax.ShapeDtypeStruct((_NUM_ITEMS, _D), _f32),
              _tab_t, _tab_t, _tab_t, _tab_t),
    mesh=plsc.VectorSubcoreMesh(core_axis_name="c", subcore_axis_name="s"),
    compiler_params=pltpu.CompilerParams(use_tc_tiling_on_sc=False,
                                         needs_layout_passes=False),
    scratch_types=[
        pltpu.VMEM_SHARED((_NPAD, _DC), _f32),
        pltpu.VMEM((_SB * _B,), jnp.int32),
        pltpu.VMEM((_SB, _B), jnp.int32),
        pltpu.VMEM((_SB * _B,), _f32),
        pltpu.VMEM((_B, _DC), _f32),
        pltpu.VMEM((_B, _DC), _f32),
        pltpu.VMEM((_B, _DC), _f32),
        pltpu.VMEM((_B, _DC), _f32),
        pltpu.VMEM((_MSLAB, _D), _f32),
        pltpu.VMEM((_MSLAB, _DC), _f32),
        pltpu.SemaphoreType.DMA,
        pltpu.SemaphoreType.DMA,
        pltpu.SemaphoreType.DMA,
        pltpu.SemaphoreType.DMA,
        pltpu.SemaphoreType.DMA,
        pltpu.SemaphoreType.DMA,
        pltpu.SemaphoreType.DMA,
        pltpu.SemaphoreType.DMA,
    ],
)


@jax.jit
def kernel(user_emb, item_emb, edge_weight, edge_index):
    src = edge_index[0].astype(jnp.int32)
    dst = edge_index[1].astype(jnp.int32)
    w = edge_weight.astype(_f32)
    pad = _EPAD - _E
    src = jnp.concatenate([src, jnp.zeros((pad,), jnp.int32)])
    dst = jnp.concatenate([dst, jnp.zeros((pad,), jnp.int32)])
    dst2 = dst.reshape(_EPAD // _B, _B)
    w = jnp.concatenate([w, jnp.zeros((pad,), _f32)])
    zh = jnp.zeros((_NPAD, _DC), _f32)
    res = _gcn(user_emb, item_emb, src, dst2, w, zh)
    return res[0], res[1]


# speedup vs baseline: 1.0561x; 1.0478x over previous
import jax
import jax.numpy as jnp
from jax import lax
from jax.experimental import pallas as pl
from jax.experimental.pallas import tpu as pltpu
from jax.experimental.pallas import tpu_sc as plsc

_NUM_USERS = 60000
_NUM_ITEMS = 40000
_N = _NUM_USERS + _NUM_ITEMS
_NPAD = 102400
_D = 64
_DC = 16
_NCHUNK = 4
_NLAYERS = 3

_E = 1_000_000
_B = 128
_NS = 16
_SB = 16
_NSUP = 31
_NBLK = _SB * _NSUP
_EPT = _B * _NBLK
_EPAD = _EPT * _NS

_RPT = _NPAD // _NS
_MSLAB = 100
_SPT = _RPT // _MSLAB

_f32 = jnp.float32


def _body(ueh, ieh, srch, dsth2, wh, zh,
          uoh, ioh, e0, e1, e2, e3,
          acc, srcv, dstv2, wv, rows0, rows1, rows2, rows3,
          prebuf, mo,
          gsem0, gsem1, gsem2, gsem3, ssem0, ssem1, ssem2, ssem3):
    cid = lax.axis_index("c")
    sid = lax.axis_index("s")
    tbase = sid * _EPT
    row0 = sid * _RPT

    tables = [e0, e1, e2, e3]
    rbufs = (rows0, rows1, rows2, rows3)
    gsems = (gsem0, gsem1, gsem2, gsem3)
    ssems = (ssem0, ssem1, ssem2, ssem3)

    ku = jnp.clip(600 - _SPT * sid, 0, _SPT)
    ke = jnp.clip(1000 - _SPT * sid, 0, _SPT)
    ccol = 2 * cid * _DC

    def _pre_pair(k2, src_tab, roff):
        g = [None, None]
        w = [None] * 4
        for h in range(2):
            r = row0 + (2 * k2 + h) * _MSLAB
            g[h] = pltpu.async_copy(
                src_tab.at[pl.ds(r + roff, _MSLAB), pl.ds(ccol, 2 * _DC)],
                prebuf.at[:, pl.ds(h * 2 * _DC, 2 * _DC)], gsems[h])
        for h in range(2):
            r = row0 + (2 * k2 + h) * _MSLAB
            g[h].wait()
            for st in range(2):
                cc = 2 * cid + st
                w[2 * h + st] = pltpu.async_copy(
                    prebuf.at[:, pl.ds((2 * h + st) * _DC, _DC)],
                    e0.at[pl.ds(cc * _NPAD + r, _MSLAB)], ssems[2 * h + st])
        for x in w:
            x.wait()

    def _pre_u(k2, c):
        _pre_pair(k2, ueh, 0)
        return c

    def _pre_i(k2, c):
        _pre_pair(k2, ieh, -_NUM_USERS)
        return c

    lax.fori_loop(0, ku // 2, _pre_u, 0)
    lax.fori_loop(ku // 2, ke // 2, _pre_i, 0)
    plsc.subcore_barrier()

    def _edge_pass(src_tab, ccoff):
        tab = src_tab.at[pl.ds(ccoff, _NPAD)]

        def _sup(s, carry):
            off = tbase + s * (_SB * _B)
            blk0 = sid * _NBLK + s * _SB
            pltpu.sync_copy(srch.at[pl.ds(off, _SB * _B)], srcv)
            pltpu.sync_copy(dsth2.at[pl.ds(blk0, _SB)], dstv2)
            pltpu.sync_copy(wh.at[pl.ds(off, _SB * _B)], wv)

            gd = [None] * _SB
            sd = [None] * _SB
            for j in range(3):
                gd[j] = pltpu.async_copy(
                    tab.at[srcv.at[pl.ds(j * _B, _B)]], rbufs[j], gsems[j])
            for j in range(_SB):
                b = j % 4
                buf = rbufs[b]
                gd[j].wait()

                def _mul(i, c2, j=j, buf=buf):
                    widx = jnp.full((_DC,), j * _B + i, jnp.int32)
                    wsplat = plsc.load_gather(wv, [widx])
                    buf[i, :] = buf[i, :] * wsplat
                    return c2

                lax.fori_loop(0, _B, _mul, 0, unroll=4)
                sd[j] = pltpu.async_copy(
                    buf, acc.at[dstv2.at[j]], ssems[b], add=True)
                nj = j + 3
                if nj < _SB:
                    if j >= 1:
                        sd[j - 1].wait()
                    gd[nj] = pltpu.async_copy(
                        tab.at[srcv.at[pl.ds(nj * _B, _B)]],
                        rbufs[nj % 4], gsems[nj % 4])
            for j in range(_SB - 3, _SB):
                sd[j].wait()
            return carry

        lax.fori_loop(0, _NSUP, _sup, 0)

    for layer in range(_NLAYERS):
        src_tab = tables[layer]
        dst_tab = tables[layer + 1]

        def _pass(step, carry, src_tab=src_tab, dst_tab=dst_tab):
            ccoff = (2 * cid + step) * _NPAD
            pltpu.sync_copy(zh.at[pl.ds(row0, _RPT)],
                            acc.at[pl.ds(row0, _RPT)])
            plsc.subcore_barrier()
            _edge_pass(src_tab, ccoff)
            plsc.subcore_barrier()
            pltpu.sync_copy(acc.at[pl.ds(row0, _RPT)],
                            dst_tab.at[pl.ds(ccoff + row0, _RPT)])
            plsc.subcore_barrier()
            return carry

        lax.fori_loop(0, 2, _pass, 0)

    quarter = _f32(0.25)

    def _mean(step, carry):
        cc = 2 * cid + step
        ccoff = cc * _NPAD
        col = cc * _DC

        def _mk_pair(k2, out, roff):
            wr = [None, None]
            for h in range(2):
                r = row0 + (2 * k2 + h) * _MSLAB
                gd = [pltpu.async_copy(
                          tables[l].at[pl.ds(ccoff + r, _MSLAB)],
                          prebuf.at[:, pl.ds(l * _DC, _DC)], gsems[l])
                      for l in range(4)]
                for d in gd:
                    d.wait()

                def _mb(i, c3, h=h):
                    mo[i, pl.ds(h * _DC, _DC)] = (
                        prebuf[i, pl.ds(0, _DC)]
                        + prebuf[i, pl.ds(_DC, _DC)]
                        + prebuf[i, pl.ds(2 * _DC, _DC)]
                        + prebuf[i, pl.ds(3 * _DC, _DC)]) * quarter
                    return c3

                lax.fori_loop(0, _MSLAB, _mb, 0, unroll=4)
                wr[h] = pltpu.async_copy(
                    mo.at[:, pl.ds(h * _DC, _DC)],
                    out.at[pl.ds(r + roff, _MSLAB), pl.ds(col, _DC)],
                    ssems[h])
            wr[0].wait()
            wr[1].wait()

        def _mk_u(k2, c2):
            _mk_pair(k2, uoh, 0)
            return c2

        def _mk_i(k2, c2):
            _mk_pair(k2, ioh, -_NUM_USERS)
            return c2

        lax.fori_loop(0, ku // 2, _mk_u, 0)
        lax.fori_loop(ku // 2, ke // 2, _mk_i, 0)
        return carry

    lax.fori_loop(0, 2, _mean, 0)


_tab_t = jax.ShapeDtypeStruct((_NCHUNK * _NPAD, _DC), _f32)

_gcn = pl.kernel(
    _body,
    out_type=(jax.ShapeDtypeStruct((_NUM_USERS, _D), _f32),
              jax.ShapeDtypeStruct((_NUM_ITEMS, _D), _f32),
              _tab_t, _tab_t, _tab_t, _tab_t),
    mesh=plsc.VectorSubcoreMesh(core_axis_name="c", subcore_axis_name="s"),
    compiler_params=pltpu.CompilerParams(use_tc_tiling_on_sc=False,
                                         needs_layout_passes=False),
    scratch_types=[
        pltpu.VMEM_SHARED((_NPAD, _DC), _f32),
        pltpu.VMEM((_SB * _B,), jnp.int32),
        pltpu.VMEM((_SB, _B), jnp.int32),
        pltpu.VMEM((_SB * _B,), _f32),
        pltpu.VMEM((_B, _DC), _f32),
        pltpu.VMEM((_B, _DC), _f32),
        pltpu.VMEM((_B, _DC), _f32),
        pltpu.VMEM((_B, _DC), _f32),
        pltpu.VMEM((_MSLAB, _D), _f32),
        pltpu.VMEM((_MSLAB, 2 * _DC), _f32),
        pltpu.SemaphoreType.DMA,
        pltpu.SemaphoreType.DMA,
        pltpu.SemaphoreType.DMA,
        pltpu.SemaphoreType.DMA,
        pltpu.SemaphoreType.DMA,
        pltpu.SemaphoreType.DMA,
        pltpu.SemaphoreType.DMA,
        pltpu.SemaphoreType.DMA,
    ],
)


@jax.jit
def kernel(user_emb, item_emb, edge_weight, edge_index):
    src = edge_index[0].astype(jnp.int32)
    dst = edge_index[1].astype(jnp.int32)
    w = edge_weight.astype(_f32)
    pad = _EPAD - _E
    src = jnp.concatenate([src, jnp.zeros((pad,), jnp.int32)])
    dst = jnp.concatenate([dst, jnp.zeros((pad,), jnp.int32)])
    dst2 = dst.reshape(_EPAD // _B, _B)
    w = jnp.concatenate([w, jnp.zeros((pad,), _f32)])
    zh = jnp.zeros((_NPAD, _DC), _f32)
    res = _gcn(user_emb, item_emb, src, dst2, w, zh)
    return res[0], res[1]
